# baseline (speedup 1.0000x reference)
import functools

import jax
import jax.numpy as jnp
from jax.experimental import pallas as pl
from jax.experimental.pallas import tpu as pltpu
from jax.experimental.pallas import tpu_sc as plsc

_K = 1024
_D = 64
_BETA = 0.25
_BLK = 2048
_CHUNKS = 4
_N_ROWS = 32768


def _tc_body(first, last, *refs):
    if first:
        z_ref, emb_ref = refs[:2]
        carry_counts_ref = carry_loss_ref = None
        refs = refs[2:]
    else:
        z_ref, emb_ref, carry_counts_ref, carry_loss_ref = refs[:4]
        refs = refs[4:]
    idx_ref = refs[0]
    refs = refs[1:]
    if first:
        embn_ref = refs[0]
        refs = refs[1:]
    if last:
        cb_ref, vq_ref, perp_ref = refs[:3]
        refs = refs[3:]
    else:
        counts_out_ref, loss_out_ref = refs[:2]
        refs = refs[2:]
    ewn16_scr, e2_scr, counts_scr, loss_scr = refs

    i = pl.program_id(0)
    nsteps = pl.num_programs(0)

    @pl.when(i == 0)
    def _init():
        ew = emb_ref[...]
        n = jnp.sqrt(jnp.sum(ew * ew, axis=1, keepdims=True))
        ewn = ew / jnp.maximum(n, 1e-12)
        if first:
            embn_ref[...] = jnp.concatenate(
                [ewn, jnp.zeros((_K, 128 - _D), jnp.float32)], axis=1)
        ewn16_scr[...] = (2.0 * ewn).astype(jnp.bfloat16)
        e2_scr[...] = jnp.sum(ewn * ewn, axis=1, keepdims=True)
        if first:
            counts_scr[...] = jnp.zeros((_K, 1), jnp.float32)
            loss_scr[...] = jnp.zeros((1, 1), jnp.float32)
        else:
            counts_scr[...] = carry_counts_ref[...]
            loss_scr[...] = carry_loss_ref[...]

    z = z_ref[...]
    zn = z / jnp.maximum(jnp.sqrt(jnp.sum(z * z, axis=1, keepdims=True)), 1e-12)
    dot2 = jax.lax.dot_general(
        ewn16_scr[...], zn.astype(jnp.bfloat16),
        (((1,), (1,)), ((), ())), preferred_element_type=jnp.float32)
    score = e2_scr[...] - dot2
    m = jnp.min(score, axis=0, keepdims=True)
    mask = score == m
    iota = jax.lax.broadcasted_iota(jnp.int32, score.shape, 0)
    idx = jnp.min(jnp.where(mask, iota, _K), axis=0, keepdims=True)
    idx_ref[...] = idx

    counts_scr[...] += jnp.sum(mask.astype(jnp.float32), axis=1, keepdims=True)
    znorm2 = jnp.sum(zn * zn, axis=1, keepdims=True)
    loss_scr[...] += jnp.reshape(jnp.sum(znorm2) + jnp.sum(m), (1, 1))

    @pl.when(i == nsteps - 1)
    def _fini():
        if last:
            cb = loss_scr[...] * (1.0 / (_N_ROWS * _D))
            p = counts_scr[...] * (1.0 / _N_ROWS)
            ent = -jnp.sum(p * jnp.log(p + 1e-10))
            cb_ref[...] = cb
            vq_ref[...] = cb + _BETA * cb
            perp_ref[...] = jnp.exp(ent) * jnp.ones((1, 1), jnp.float32)
        else:
            counts_out_ref[...] = counts_scr[...]
            loss_out_ref[...] = loss_scr[...]


def _tc_stage(z_chunk, emb_weight, carry, first, last):
    rows = z_chunk.shape[0]
    grid = rows // _BLK
    in_specs = [
        pl.BlockSpec((_BLK, _D), lambda i: (i, 0)),
        pl.BlockSpec((_K, _D), lambda i: (0, 0)),
    ]
    args = [z_chunk, emb_weight]
    if not first:
        in_specs += [pl.BlockSpec((_K, 1), lambda i: (0, 0)),
                     pl.BlockSpec((1, 1), lambda i: (0, 0))]
        args += [carry[0], carry[1]]
    out_specs = [pl.BlockSpec((1, _BLK), lambda i: (0, i))]
    out_shape = [jax.ShapeDtypeStruct((1, rows), jnp.int32)]
    if first:
        out_specs += [pl.BlockSpec((_K, 128), lambda i: (0, 0))]
        out_shape += [jax.ShapeDtypeStruct((_K, 128), jnp.float32)]
    if last:
        out_specs += [pl.BlockSpec((1, 1), lambda i: (0, 0))] * 3
        out_shape += [jax.ShapeDtypeStruct((1, 1), jnp.float32)] * 3
    else:
        out_specs += [pl.BlockSpec((_K, 1), lambda i: (0, 0)),
                      pl.BlockSpec((1, 1), lambda i: (0, 0))]
        out_shape += [jax.ShapeDtypeStruct((_K, 1), jnp.float32),
                      jax.ShapeDtypeStruct((1, 1), jnp.float32)]
    return pl.pallas_call(
        functools.partial(_tc_body, first, last),
        grid=(grid,),
        in_specs=in_specs,
        out_specs=out_specs,
        out_shape=out_shape,
        scratch_shapes=[
            pltpu.VMEM((_K, _D), jnp.bfloat16),
            pltpu.VMEM((_K, 1), jnp.float32),
            pltpu.VMEM((_K, 1), jnp.float32),
            pltpu.VMEM((1, 1), jnp.float32),
        ],
    )(*args)


_GATHER_WIN = 128


def _sc_gather(table, indices_2d):
    n_rows = indices_2d.shape[1]
    mesh = plsc.VectorSubcoreMesh(core_axis_name="core",
                                  subcore_axis_name="subcore")

    @pl.kernel(out_type=jax.ShapeDtypeStruct((n_rows, _D), table.dtype),
               mesh=mesh,
               scratch_types=[pltpu.VMEM((_GATHER_WIN, 128), jnp.float32)])
    def k(x_hbm, i_hbm, o_hbm, pad_scr):
        def body(i_vmem, o_vmem):
            pltpu.sync_copy(x_hbm.at[i_vmem.at[0]], pad_scr)
            o_vmem[...] = pad_scr[:, :_D]

        pltpu.emit_pipeline(
            body,
            grid=(n_rows // _GATHER_WIN,),
            in_specs=[pl.BlockSpec((1, _GATHER_WIN), index_map=lambda i: (0, i))],
            out_specs=[pl.BlockSpec((_GATHER_WIN, _D), index_map=lambda i: (i, 0))],
            core_axis_name=("core", "subcore"),
            dimension_semantics=(pltpu.PARALLEL,),
        )(i_hbm, o_hbm)

    return k(table, indices_2d)


def kernel(z_e, emb_weight):
    rows_per = _N_ROWS // _CHUNKS
    z_flat = z_e.reshape(_N_ROWS, _D)

    idx_chunks, zq_chunks = [], []
    carry = None
    embn = None
    scalars = None
    for c in range(_CHUNKS):
        z_chunk = jax.lax.slice_in_dim(z_flat, c * rows_per, (c + 1) * rows_per)
        outs = _tc_stage(z_chunk, emb_weight, carry,
                         first=(c == 0), last=(c == _CHUNKS - 1))
        idx_c = outs[0]
        rest = outs[1:]
        if c == 0:
            embn = rest[0]
            rest = rest[1:]
        if c == _CHUNKS - 1:
            scalars = rest
        else:
            carry = rest
        idx_chunks.append(idx_c)
        zq_chunks.append(_sc_gather(embn, idx_c))

    z_q = jnp.concatenate(zq_chunks, axis=0).reshape(z_e.shape)
    idx = jnp.concatenate(idx_chunks, axis=1).reshape(_N_ROWS)
    cb, vq, perp = scalars
    cb_s = cb[0, 0]
    return (z_q, vq[0, 0], cb_s, cb_s, perp[0, 0], idx)


# speedup vs baseline: 2.0484x; 2.0484x over previous
import jax
import jax.numpy as jnp
from jax.experimental import pallas as pl
from jax.experimental.pallas import tpu as pltpu

_K = 1024
_D = 64
_BETA = 0.25
_B = 1024


def _body(z_ref, emb_ref, zq_ref, idx_ref, cb_ref, vq_ref, perp_ref,
          ew2_scr, whilo_scr, e2_scr, counts_scr, loss_scr):
    i = pl.program_id(0)
    nsteps = pl.num_programs(0)

    @pl.when(i == 0)
    def _init():
        ew = emb_ref[...]
        n = jnp.sqrt(jnp.sum(ew * ew, axis=1, keepdims=True))
        ewn = ew / jnp.maximum(n, 1e-12)
        ew2_scr[...] = (2.0 * ewn).astype(jnp.bfloat16)
        hi = ewn.astype(jnp.bfloat16)
        lo = (ewn - hi.astype(jnp.float32)).astype(jnp.bfloat16)
        whilo_scr[...] = jnp.concatenate([hi, lo], axis=1)
        e2_scr[...] = jnp.sum(ewn * ewn, axis=1, keepdims=True)
        counts_scr[...] = jnp.zeros((_K, 1), jnp.float32)
        loss_scr[...] = jnp.zeros((1, 1), jnp.float32)

    z = jnp.swapaxes(z_ref[...], 0, 1)
    zn = z / jnp.maximum(jnp.sqrt(jnp.sum(z * z, axis=1, keepdims=True)), 1e-12)
    dot2 = jax.lax.dot_general(
        ew2_scr[...], zn.astype(jnp.bfloat16),
        (((1,), (1,)), ((), ())), preferred_element_type=jnp.float32)
    score = e2_scr[...] - dot2
    m = jnp.min(score, axis=0, keepdims=True)
    mask = score == m
    iota = jax.lax.broadcasted_iota(jnp.int32, score.shape, 0)
    idx = jnp.min(jnp.where(mask, iota, _K), axis=0, keepdims=True)
    idx_ref[...] = idx

    onehot = jnp.where(iota == idx, 1.0, 0.0).astype(jnp.bfloat16)
    zq2 = jax.lax.dot_general(
        whilo_scr[...], onehot,
        (((0,), (0,)), ((), ())), preferred_element_type=jnp.float32)
    zq_ref[...] = zq2[:_D, :] + zq2[_D:, :]

    counts_scr[...] += jnp.sum(onehot.astype(jnp.float32), axis=1, keepdims=True)
    znorm2 = jnp.sum(zn * zn, axis=1, keepdims=True)
    loss_scr[...] += jnp.reshape(jnp.sum(znorm2) + jnp.sum(m), (1, 1))

    @pl.when(i == nsteps - 1)
    def _fini():
        total = nsteps * _B
        cb = loss_scr[...] * (1.0 / (total * _D))
        p = counts_scr[...] * (1.0 / total)
        ent = -jnp.sum(p * jnp.log(p + 1e-10))
        cb_ref[...] = cb
        vq_ref[...] = cb + _BETA * cb
        perp_ref[...] = jnp.exp(ent) * jnp.ones((1, 1), jnp.float32)


def kernel(z_e, emb_weight):
    nbatch, ntok, _ = z_e.shape
    n_rows = nbatch * ntok
    z2d = jnp.transpose(z_e, (0, 2, 1)).reshape(nbatch * _D, ntok)

    zq2d, idx, cb, vq, perp = pl.pallas_call(
        _body,
        grid=(nbatch,),
        in_specs=[
            pl.BlockSpec((_D, _B), lambda i: (i, 0)),
            pl.BlockSpec((_K, _D), lambda i: (0, 0)),
        ],
        out_specs=[
            pl.BlockSpec((_D, _B), lambda i: (i, 0)),
            pl.BlockSpec((1, _B), lambda i: (0, i)),
            pl.BlockSpec((1, 1), lambda i: (0, 0)),
            pl.BlockSpec((1, 1), lambda i: (0, 0)),
            pl.BlockSpec((1, 1), lambda i: (0, 0)),
        ],
        out_shape=[
            jax.ShapeDtypeStruct((nbatch * _D, ntok), jnp.float32),
            jax.ShapeDtypeStruct((1, n_rows), jnp.int32),
            jax.ShapeDtypeStruct((1, 1), jnp.float32),
            jax.ShapeDtypeStruct((1, 1), jnp.float32),
            jax.ShapeDtypeStruct((1, 1), jnp.float32),
        ],
        scratch_shapes=[
            pltpu.VMEM((_K, _D), jnp.bfloat16),
            pltpu.VMEM((_K, 2 * _D), jnp.bfloat16),
            pltpu.VMEM((_K, 1), jnp.float32),
            pltpu.VMEM((_K, 1), jnp.float32),
            pltpu.VMEM((1, 1), jnp.float32),
        ],
    )(z2d, emb_weight)

    z_q = jnp.transpose(zq2d.reshape(nbatch, _D, ntok), (0, 2, 1))
    cb_s = cb[0, 0]
    return (z_q, vq[0, 0], cb_s, cb_s, perp[0, 0], idx.reshape(n_rows))
